# Initial kernel scaffold; baseline (speedup 1.0000x reference)
import functools

import jax
import jax.numpy as jnp
from jax import lax
from jax.experimental import pallas as pl
from jax.experimental.pallas import tpu as pltpu
from jax.experimental.pallas import tpu_sc as plsc

_NC = 2
_NS = 16
_NW = _NC * _NS
_L = 16
_ZR = 0.8


def _trans_body(x_ref, w0_ref, b0_ref, g0_ref, t0_ref, w1_ref, b1_ref, g1_ref,
                t1_ref, base_ref, swap_ref):
  x = x_ref[...]

  def tr(w, b, g, t):
    h = jnp.dot(x, w, preferred_element_type=jnp.float32) + b
    m = jnp.mean(h, axis=-1, keepdims=True)
    v = jnp.mean((h - m) * (h - m), axis=-1, keepdims=True)
    h = (h - m) / jnp.sqrt(v + 1e-5) * g + t
    return jnp.maximum(h, 0.0)

  x0 = tr(w0_ref[...], b0_ref[...], g0_ref[...], t0_ref[...])
  x1 = tr(w1_ref[...], b1_ref[...], g1_ref[...], t1_ref[...])
  base_ref[...] = _ZR * x0 + (1.0 - _ZR) * x1
  swap_ref[...] = _ZR * x1 + (1.0 - _ZR) * x0


def _transforms(xp, w0, b0, g0, t0, w1, b1, g1, t1, br):
  np_, c = xp.shape
  grid = (np_ // br,)
  row_spec = pl.BlockSpec((br, c), lambda i: (i, 0))
  mat_spec = pl.BlockSpec((c, c), lambda i: (0, 0))
  vec_spec = pl.BlockSpec((1, c), lambda i: (0, 0))
  out_sds = jax.ShapeDtypeStruct((np_, c), jnp.float32)
  return pl.pallas_call(
      _trans_body,
      grid=grid,
      in_specs=[row_spec, mat_spec, vec_spec, vec_spec, vec_spec,
                mat_spec, vec_spec, vec_spec, vec_spec],
      out_specs=[row_spec, row_spec],
      out_shape=[out_sds, out_sds],
  )(xp, w0, b0.reshape(1, c), g0.reshape(1, c), t0.reshape(1, c),
    w1, b1.reshape(1, c), g1.reshape(1, c), t1.reshape(1, c))


def _mix_overwrite(base, swap, r_pad, c_pad, rpt):
  np_, c = base.shape
  ngrp = r_pad.shape[0] // _L
  mesh = plsc.VectorSubcoreMesh(core_axis_name="c", subcore_axis_name="s")

  @functools.partial(
      pl.kernel,
      out_type=jax.ShapeDtypeStruct((np_, c), jnp.float32),
      mesh=mesh,
      scratch_types=[
          pltpu.VMEM((rpt, c), jnp.float32),
          pltpu.VMEM((rpt, c), jnp.float32),
          pltpu.VMEM((r_pad.shape[0],), jnp.int32),
          pltpu.VMEM((r_pad.shape[0],), jnp.int32),
      ],
  )
  def k(base_hbm, swap_hbm, r_hbm, c_hbm, out_hbm, xloc, sloc, rbuf, cbuf):
    cid = lax.axis_index("c")
    sid = lax.axis_index("s")
    wid = cid * _NS + sid
    row0 = wid * rpt
    pltpu.sync_copy(base_hbm.at[pl.ds(row0, rpt)], xloc)
    pltpu.sync_copy(swap_hbm.at[pl.ds(row0, rpt)], sloc)
    pltpu.sync_copy(r_hbm, rbuf)
    pltpu.sync_copy(c_hbm, cbuf)

    def body(g, carry):
      r16 = rbuf[pl.ds(g * _L, _L)]
      c16 = cbuf[pl.ds(g * _L, _L)]
      lr = r16 - row0
      msk = (lr >= 0) & (lr < rpt)
      lrc = jnp.clip(lr, 0, rpt - 1)
      c16c = jnp.clip(c16, 0, c - 1)
      v = plsc.load_gather(sloc, [lrc, c16c], mask=msk)
      plsc.store_scatter(xloc, [lrc, c16c], v, mask=msk)
      return carry

    lax.fori_loop(0, ngrp, body, 0)
    pltpu.sync_copy(xloc, out_hbm.at[pl.ds(row0, rpt)])

  return k(base, swap, r_pad, c_pad)


def _spmm(xf, rows3, cols3, w3, np_, batch):
  c = xf.shape[1]
  nb = rows3.shape[1]
  rpt16 = np_ // _NS
  nz = rpt16 // 64
  mesh = plsc.VectorSubcoreMesh(core_axis_name="c", subcore_axis_name="s")

  @functools.partial(
      pl.kernel,
      out_type=(jax.ShapeDtypeStruct((_NC * np_, c), jnp.float32),
                jax.ShapeDtypeStruct((_NC * np_, _L), jnp.float32)),
      mesh=mesh,
      scratch_types=[
          pltpu.VMEM_SHARED((np_, c), jnp.float32),
          pltpu.VMEM_SHARED((np_, _L), jnp.float32),
          pltpu.VMEM((nb, batch), jnp.int32),
          pltpu.VMEM((nb, batch), jnp.int32),
          pltpu.VMEM((nb, batch), jnp.float32),
          pltpu.VMEM((batch, c), jnp.float32),
          pltpu.VMEM((batch, _L), jnp.float32),
          pltpu.VMEM((64, c), jnp.float32),
          pltpu.VMEM((64, _L), jnp.float32),
          pltpu.SemaphoreType.DMA,
      ],
  )
  def k(x_hbm, r_hbm, c_hbm, w_hbm, feat_hbm, cnt_hbm,
        acc_f, acc_c, rbuf, cbuf, wbuf, rowsb, ones_src, zbuf, zcnt, gsem):
    cid = lax.axis_index("c")
    sid = lax.axis_index("s")
    chunk = cid * _NS + sid

    zv = jnp.zeros((_L,), jnp.float32)

    def zb(i, carry):
      for cc in range(c // _L):
        zbuf[i, pl.ds(cc * _L, _L)] = zv
      zcnt[i, pl.ds(0, _L)] = zv
      return carry

    lax.fori_loop(0, 64, zb, 0)
    onehot = jnp.where(lax.iota(jnp.int32, _L) == 0, 1.0, 0.0)

    def ob(e, carry):
      ones_src[e, pl.ds(0, _L)] = onehot
      return carry

    lax.fori_loop(0, batch, ob, 0)
    for kk in range(nz):
      pltpu.sync_copy(zbuf, acc_f.at[pl.ds(sid * rpt16 + kk * 64, 64)])
      pltpu.sync_copy(zcnt, acc_c.at[pl.ds(sid * rpt16 + kk * 64, 64)])
    pltpu.sync_copy(r_hbm.at[chunk], rbuf)
    pltpu.sync_copy(c_hbm.at[chunk], cbuf)
    pltpu.sync_copy(w_hbm.at[chunk], wbuf)
    plsc.subcore_barrier()

    def body(i, carry):
      pltpu.async_copy(x_hbm.at[cbuf.at[i]], rowsb, gsem).wait()
      for e in range(batch):
        w = plsc.load_gather(
            wbuf, [jnp.full((_L,), i, jnp.int32), jnp.full((_L,), e, jnp.int32)])
        for cc in range(c // _L):
          rowsb[e, pl.ds(cc * _L, _L)] = rowsb[e, pl.ds(cc * _L, _L)] * w
      pltpu.sync_copy(rowsb, acc_f.at[rbuf.at[i]], add=True)
      pltpu.sync_copy(ones_src, acc_c.at[rbuf.at[i]], add=True)
      return carry

    lax.fori_loop(0, nb, body, 0)
    plsc.subcore_barrier()
    pltpu.sync_copy(acc_f.at[pl.ds(sid * rpt16, rpt16)],
                    feat_hbm.at[pl.ds(cid * np_ + sid * rpt16, rpt16)])
    pltpu.sync_copy(acc_c.at[pl.ds(sid * rpt16, rpt16)],
                    cnt_hbm.at[pl.ds(cid * np_ + sid * rpt16, rpt16)])

  return k(xf, rows3, cols3, w3)


def _out_body(f0_ref, f1_ref, c0_ref, c1_ref, w_ref, b_ref, o_ref):
  s = f0_ref[...] + f1_ref[...]
  n = c0_ref[...][:, 0:1] + c1_ref[...][:, 0:1]
  agg = s / jnp.maximum(n, 1.0)
  o_ref[...] = jnp.maximum(
      jnp.dot(agg, w_ref[...], preferred_element_type=jnp.float32) + b_ref[...],
      0.0)


def _out_linear(f0, f1, c0, c1, wout, bout, br):
  np_, c = f0.shape
  grid = (np_ // br,)
  row_spec = pl.BlockSpec((br, c), lambda i: (i, 0))
  cnt_spec = pl.BlockSpec((br, _L), lambda i: (i, 0))
  mat_spec = pl.BlockSpec((c, c), lambda i: (0, 0))
  vec_spec = pl.BlockSpec((1, c), lambda i: (0, 0))
  return pl.pallas_call(
      _out_body,
      grid=grid,
      in_specs=[row_spec, row_spec, cnt_spec, cnt_spec, mat_spec, vec_spec],
      out_specs=row_spec,
      out_shape=jax.ShapeDtypeStruct((np_, c), jnp.float32),
  )(f0, f1, c0, c1, wout, bout.reshape(1, c))


def kernel(x_, edge_index, edge_weight, n2s_row, n2s_col,
           W0, b0, g0, beta0, W1, b1, g1, beta1, Wout, bout):
  n, c = x_.shape
  e = edge_weight.shape[0]
  nnz = n2s_row.shape[0]

  rpt = ((n + _NW - 1) // _NW + 63) // 64 * 64
  np_ = rpt * _NW
  batch = 80
  assert e % (_NW * batch) == 0, e
  nb = e // (_NW * batch)

  xp = jnp.pad(x_, ((0, np_ - n), (0, 0)))
  base, swap = _transforms(xp, W0, b0, g0, beta0, W1, b1, g1, beta1, 1024)

  nnz_pad = (nnz + _L - 1) // _L * _L
  r_pad = jnp.pad(n2s_row.astype(jnp.int32), (0, nnz_pad - nnz),
                  constant_values=-1)
  c_pad = jnp.pad(n2s_col.astype(jnp.int32), (0, nnz_pad - nnz))
  xf = _mix_overwrite(base, swap, r_pad, c_pad, rpt)

  rows3 = edge_index[0].astype(jnp.int32).reshape(_NW, nb, batch)
  cols3 = edge_index[1].astype(jnp.int32).reshape(_NW, nb, batch)
  w3 = edge_weight.reshape(_NW, nb, batch)
  feat, cnt = _spmm(xf, rows3, cols3, w3, np_, batch)

  out = _out_linear(feat[:np_], feat[np_:], cnt[:np_], cnt[np_:],
                    Wout, bout, 1024)
  return out[:n]


# speedup vs baseline: 2.3305x; 2.3305x over previous
import functools

import jax
import jax.numpy as jnp
from jax import lax
from jax.experimental import pallas as pl
from jax.experimental.pallas import tpu as pltpu
from jax.experimental.pallas import tpu_sc as plsc

_NC = 2
_NS = 16
_NW = _NC * _NS
_L = 16
_ZR = 0.8


def _trans_body(x_ref, w0_ref, b0_ref, g0_ref, t0_ref, w1_ref, b1_ref, g1_ref,
                t1_ref, base_ref, swap_ref):
  x = x_ref[...]

  def tr(w, b, g, t):
    h = jnp.dot(x, w, preferred_element_type=jnp.float32) + b
    m = jnp.mean(h, axis=-1, keepdims=True)
    v = jnp.mean((h - m) * (h - m), axis=-1, keepdims=True)
    h = (h - m) / jnp.sqrt(v + 1e-5) * g + t
    return jnp.maximum(h, 0.0)

  x0 = tr(w0_ref[...], b0_ref[...], g0_ref[...], t0_ref[...])
  x1 = tr(w1_ref[...], b1_ref[...], g1_ref[...], t1_ref[...])
  base_ref[...] = _ZR * x0 + (1.0 - _ZR) * x1
  swap_ref[...] = _ZR * x1 + (1.0 - _ZR) * x0


def _transforms(xp, w0, b0, g0, t0, w1, b1, g1, t1, br):
  np_, c = xp.shape
  grid = (np_ // br,)
  row_spec = pl.BlockSpec((br, c), lambda i: (i, 0))
  mat_spec = pl.BlockSpec((c, c), lambda i: (0, 0))
  vec_spec = pl.BlockSpec((1, c), lambda i: (0, 0))
  out_sds = jax.ShapeDtypeStruct((np_, c), jnp.float32)
  return pl.pallas_call(
      _trans_body,
      grid=grid,
      in_specs=[row_spec, mat_spec, vec_spec, vec_spec, vec_spec,
                mat_spec, vec_spec, vec_spec, vec_spec],
      out_specs=[row_spec, row_spec],
      out_shape=[out_sds, out_sds],
  )(xp, w0, b0.reshape(1, c), g0.reshape(1, c), t0.reshape(1, c),
    w1, b1.reshape(1, c), g1.reshape(1, c), t1.reshape(1, c))


def _mix_overwrite(base, swap, r_pad, c_pad, rpt):
  np_, c = base.shape
  ngrp = r_pad.shape[0] // _L
  mesh = plsc.VectorSubcoreMesh(core_axis_name="c", subcore_axis_name="s")

  @functools.partial(
      pl.kernel,
      out_type=jax.ShapeDtypeStruct((np_, c), jnp.float32),
      mesh=mesh,
      compiler_params=pltpu.CompilerParams(needs_layout_passes=False),
      scratch_types=[
          pltpu.VMEM((rpt, c), jnp.float32),
          pltpu.VMEM((rpt, c), jnp.float32),
          pltpu.VMEM((r_pad.shape[0],), jnp.int32),
          pltpu.VMEM((r_pad.shape[0],), jnp.int32),
      ],
  )
  def k(base_hbm, swap_hbm, r_hbm, c_hbm, out_hbm, xloc, sloc, rbuf, cbuf):
    cid = lax.axis_index("c")
    sid = lax.axis_index("s")
    wid = cid * _NS + sid
    row0 = wid * rpt
    pltpu.sync_copy(base_hbm.at[pl.ds(row0, rpt)], xloc)
    pltpu.sync_copy(swap_hbm.at[pl.ds(row0, rpt)], sloc)
    pltpu.sync_copy(r_hbm, rbuf)
    pltpu.sync_copy(c_hbm, cbuf)

    def body(g, carry):
      r16 = rbuf[pl.ds(g * _L, _L)]
      c16 = cbuf[pl.ds(g * _L, _L)]
      lr = r16 - row0
      msk = (lr >= 0) & (lr < rpt)
      lrc = jnp.clip(lr, 0, rpt - 1)
      c16c = jnp.clip(c16, 0, c - 1)
      v = plsc.load_gather(sloc, [lrc, c16c], mask=msk)
      plsc.store_scatter(xloc, [lrc, c16c], v, mask=msk)
      return carry

    lax.fori_loop(0, ngrp, body, 0)
    pltpu.sync_copy(xloc, out_hbm.at[pl.ds(row0, rpt)])

  return k(base, swap, r_pad, c_pad)


def _spmm_t(xT3, rowf, colf, wf, np_, ch):
  ncol = xT3.shape[1]
  e = rowf.shape[0]
  nch = e // ch
  npg = ch // _L
  mesh = plsc.VectorSubcoreMesh(core_axis_name="c", subcore_axis_name="s")

  @functools.partial(
      pl.kernel,
      out_type=jax.ShapeDtypeStruct((_NW, ncol + 1, np_), jnp.float32),
      mesh=mesh,
      compiler_params=pltpu.CompilerParams(needs_layout_passes=False),
      scratch_types=[
          pltpu.VMEM((ncol, np_), jnp.float32),
          pltpu.VMEM((ncol + 1, np_), jnp.float32),
          pltpu.VMEM((ch,), jnp.int32),
          pltpu.VMEM((ch,), jnp.int32),
          pltpu.VMEM((ch,), jnp.float32),
      ],
  )
  def k(x_hbm, r_hbm, c_hbm, w_hbm, out_hbm, xs, acc, rbuf, cbuf, wbuf):
    cid = lax.axis_index("c")
    sid = lax.axis_index("s")
    wid = cid * _NS + sid
    pltpu.sync_copy(x_hbm.at[wid], xs)

    zv = jnp.zeros((_L,), jnp.float32)

    def zb(i, carry):
      for j in range(ncol + 1):
        acc[j, pl.ds(i * _L, _L)] = zv
      return carry

    lax.fori_loop(0, np_ // _L, zb, 0)

    ones16 = jnp.full((_L,), 1.0, jnp.float32)
    cnt_row = jnp.full((_L,), ncol, jnp.int32)

    def grp(g, carry):
      r16 = rbuf[pl.ds(g * _L, _L)]
      c16 = cbuf[pl.ds(g * _L, _L)]
      w16 = wbuf[pl.ds(g * _L, _L)]
      for j in range(ncol):
        jv = jnp.full((_L,), j, jnp.int32)
        v = plsc.load_gather(xs, [jv, c16])
        plsc.addupdate_scatter(acc, [jv, r16], v * w16)
      plsc.addupdate_scatter(acc, [cnt_row, r16], ones16)
      return carry

    def chunk(i, carry):
      kc = (i + wid * (nch // _NW)) % nch
      pltpu.sync_copy(r_hbm.at[pl.ds(kc * ch, ch)], rbuf)
      pltpu.sync_copy(c_hbm.at[pl.ds(kc * ch, ch)], cbuf)
      pltpu.sync_copy(w_hbm.at[pl.ds(kc * ch, ch)], wbuf)
      lax.fori_loop(0, npg, grp, 0)
      return carry

    lax.fori_loop(0, nch, chunk, 0)
    pltpu.sync_copy(acc, out_hbm.at[wid])

  return k(xT3, rowf, colf, wf)


def _out_body(ft_ref, cnt_ref, w_ref, b_ref, o_ref):
  aggt = ft_ref[...] / jnp.maximum(cnt_ref[...], 1.0)
  o_ref[...] = jnp.maximum(
      jax.lax.dot_general(aggt, w_ref[...], (((0,), (0,)), ((), ())),
                          preferred_element_type=jnp.float32) + b_ref[...],
      0.0)


def _out_linear(ft, cnt, wout, bout, br):
  c, np_ = ft.shape
  grid = (np_ // br,)
  return pl.pallas_call(
      _out_body,
      grid=grid,
      in_specs=[pl.BlockSpec((c, br), lambda i: (0, i)),
                pl.BlockSpec((1, br), lambda i: (0, i)),
                pl.BlockSpec((c, c), lambda i: (0, 0)),
                pl.BlockSpec((1, c), lambda i: (0, 0))],
      out_specs=pl.BlockSpec((br, c), lambda i: (i, 0)),
      out_shape=jax.ShapeDtypeStruct((np_, c), jnp.float32),
  )(ft, cnt, wout, bout.reshape(1, c))


def kernel(x_, edge_index, edge_weight, n2s_row, n2s_col,
           W0, b0, g0, beta0, W1, b1, g1, beta1, Wout, bout):
  n, c = x_.shape
  e = edge_weight.shape[0]
  nnz = n2s_row.shape[0]

  rpt = ((n + _NW - 1) // _NW + 63) // 64 * 64
  np_ = rpt * _NW
  ch = 2000
  assert e % (ch * _NW) == 0, e

  xp = jnp.pad(x_, ((0, np_ - n), (0, 0)))
  base, swap = _transforms(xp, W0, b0, g0, beta0, W1, b1, g1, beta1, 1024)

  nnz_pad = (nnz + _L - 1) // _L * _L
  r_pad = jnp.pad(n2s_row.astype(jnp.int32), (0, nnz_pad - nnz),
                  constant_values=-1)
  c_pad = jnp.pad(n2s_col.astype(jnp.int32), (0, nnz_pad - nnz))
  xf = _mix_overwrite(base, swap, r_pad, c_pad, rpt)

  xT3 = jnp.transpose(xf).reshape(_NW, c // _NW, np_)
  rowf = edge_index[0].astype(jnp.int32)
  colf = edge_index[1].astype(jnp.int32)
  acc = _spmm_t(xT3, rowf, colf, edge_weight, np_, ch)

  ft = acc[:, : c // _NW, :].reshape(c, np_)
  cnt = acc[0, c // _NW, :].reshape(1, np_)
  out = _out_linear(ft, cnt, Wout, bout, 1024)
  return out[:n]


# speedup vs baseline: 2.8967x; 1.2429x over previous
import functools

import jax
import jax.numpy as jnp
from jax import lax
from jax.experimental import pallas as pl
from jax.experimental.pallas import tpu as pltpu
from jax.experimental.pallas import tpu_sc as plsc

_NC = 2
_NS = 16
_NW = _NC * _NS
_L = 16
_ZR = 0.8


def _trans_body(x_ref, w0_ref, b0_ref, g0_ref, t0_ref, w1_ref, b1_ref, g1_ref,
                t1_ref, base_ref, swap_ref):
  x = x_ref[...]

  def tr(w, b, g, t):
    h = jnp.dot(x, w, preferred_element_type=jnp.float32) + b
    m = jnp.mean(h, axis=-1, keepdims=True)
    v = jnp.mean((h - m) * (h - m), axis=-1, keepdims=True)
    h = (h - m) / jnp.sqrt(v + 1e-5) * g + t
    return jnp.maximum(h, 0.0)

  x0 = tr(w0_ref[...], b0_ref[...], g0_ref[...], t0_ref[...])
  x1 = tr(w1_ref[...], b1_ref[...], g1_ref[...], t1_ref[...])
  base_ref[...] = _ZR * x0 + (1.0 - _ZR) * x1
  swap_ref[...] = _ZR * x1 + (1.0 - _ZR) * x0


def _transforms(xp, w0, b0, g0, t0, w1, b1, g1, t1, br):
  np_, c = xp.shape
  grid = (np_ // br,)
  row_spec = pl.BlockSpec((br, c), lambda i: (i, 0))
  mat_spec = pl.BlockSpec((c, c), lambda i: (0, 0))
  vec_spec = pl.BlockSpec((1, c), lambda i: (0, 0))
  out_sds = jax.ShapeDtypeStruct((np_, c), jnp.float32)
  return pl.pallas_call(
      _trans_body,
      grid=grid,
      in_specs=[row_spec, mat_spec, vec_spec, vec_spec, vec_spec,
                mat_spec, vec_spec, vec_spec, vec_spec],
      out_specs=[row_spec, row_spec],
      out_shape=[out_sds, out_sds],
  )(xp, w0, b0.reshape(1, c), g0.reshape(1, c), t0.reshape(1, c),
    w1, b1.reshape(1, c), g1.reshape(1, c), t1.reshape(1, c))


def _mix_overwrite(base, swap, r_pad, c_pad, rpt):
  np_, c = base.shape
  ngrp = r_pad.shape[0] // _L
  mesh = plsc.VectorSubcoreMesh(core_axis_name="c", subcore_axis_name="s")

  @functools.partial(
      pl.kernel,
      out_type=jax.ShapeDtypeStruct((np_, c), jnp.float32),
      mesh=mesh,
      compiler_params=pltpu.CompilerParams(needs_layout_passes=False),
      scratch_types=[
          pltpu.VMEM((rpt, c), jnp.float32),
          pltpu.VMEM((rpt, c), jnp.float32),
          pltpu.VMEM((r_pad.shape[0],), jnp.int32),
          pltpu.VMEM((r_pad.shape[0],), jnp.int32),
      ],
  )
  def k(base_hbm, swap_hbm, r_hbm, c_hbm, out_hbm, xloc, sloc, rbuf, cbuf):
    cid = lax.axis_index("c")
    sid = lax.axis_index("s")
    wid = cid * _NS + sid
    row0 = wid * rpt
    pltpu.sync_copy(base_hbm.at[pl.ds(row0, rpt)], xloc)
    pltpu.sync_copy(swap_hbm.at[pl.ds(row0, rpt)], sloc)
    pltpu.sync_copy(r_hbm, rbuf)
    pltpu.sync_copy(c_hbm, cbuf)

    def body(g, carry):
      r16 = rbuf[pl.ds(g * _L, _L)]
      c16 = cbuf[pl.ds(g * _L, _L)]
      lr = r16 - row0
      msk = (lr >= 0) & (lr < rpt)
      lrc = jnp.clip(lr, 0, rpt - 1)
      c16c = jnp.clip(c16, 0, c - 1)
      v = plsc.load_gather(sloc, [lrc, c16c], mask=msk)
      plsc.store_scatter(xloc, [lrc, c16c], v, mask=msk)
      return carry

    lax.fori_loop(0, ngrp, body, 0)
    pltpu.sync_copy(xloc, out_hbm.at[pl.ds(row0, rpt)])

  return k(base, swap, r_pad, c_pad)


def _spmm_t(xT3, meta, np_, ch):
  ncol = xT3.shape[1]
  nch = meta.shape[0] // (3 * ch)
  npg = ch // _L
  mesh = plsc.VectorSubcoreMesh(core_axis_name="c", subcore_axis_name="s")

  @functools.partial(
      pl.kernel,
      out_type=jax.ShapeDtypeStruct((_NW, ncol + 1, np_), jnp.float32),
      mesh=mesh,
      compiler_params=pltpu.CompilerParams(needs_layout_passes=False),
      scratch_types=[
          pltpu.VMEM((ncol, np_), jnp.float32),
          pltpu.VMEM((ncol + 1, np_), jnp.float32),
          pltpu.VMEM((3 * ch,), jnp.int32),
      ],
  )
  def k(x_hbm, m_hbm, out_hbm, xs, acc, mbuf):
    cid = lax.axis_index("c")
    sid = lax.axis_index("s")
    wid = cid * _NS + sid
    pltpu.sync_copy(x_hbm.at[wid], xs)

    zv = jnp.zeros((_L,), jnp.float32)

    def zb(i, carry):
      for j in range(ncol + 1):
        acc[j, pl.ds(i * _L, _L)] = zv
      return carry

    lax.fori_loop(0, np_ // _L, zb, 0)

    ones16 = jnp.full((_L,), 1.0, jnp.float32)
    cnt_row = jnp.full((_L,), ncol, jnp.int32)

    def make_grp(cmask):
      def grp(g, carry):
        r16 = mbuf[pl.ds(g * _L, _L)]
        c16 = mbuf[pl.ds(ch + g * _L, _L)]
        w16 = plsc.bitcast(mbuf[pl.ds(2 * ch + g * _L, _L)], jnp.float32)
        for j in range(ncol):
          jv = jnp.full((_L,), j, jnp.int32)
          v = plsc.load_gather(xs, [jv, c16])
          plsc.addupdate_scatter(acc, [jv, r16], v * w16)
        plsc.addupdate_scatter(acc, [cnt_row, r16], ones16, mask=cmask)
        return carry
      return grp

    def chunk(i, carry):
      kc = (i + wid) % nch
      pltpu.sync_copy(m_hbm.at[pl.ds(kc * 3 * ch, 3 * ch)], mbuf)
      cmask = jnp.full((_L,), True) & (jnp.full((_L,), kc % _NW) == wid)
      lax.fori_loop(0, npg, make_grp(cmask), 0, unroll=4)
      return carry

    lax.fori_loop(0, nch, chunk, 0)
    pltpu.sync_copy(acc, out_hbm.at[wid])

  return k(xT3, meta)


def _out_body(ft_ref, cnt_ref, w_ref, b_ref, o_ref):
  n = jnp.sum(cnt_ref[...], axis=0, keepdims=True)
  aggt = ft_ref[...] / jnp.maximum(n, 1.0)
  o_ref[...] = jnp.maximum(
      jax.lax.dot_general(aggt, w_ref[...], (((0,), (0,)), ((), ())),
                          preferred_element_type=jnp.float32) + b_ref[...],
      0.0)


def _out_linear(ft, cnt, wout, bout, br):
  c, np_ = ft.shape
  grid = (np_ // br,)
  return pl.pallas_call(
      _out_body,
      grid=grid,
      in_specs=[pl.BlockSpec((c, br), lambda i: (0, i)),
                pl.BlockSpec((_NW, br), lambda i: (0, i)),
                pl.BlockSpec((c, c), lambda i: (0, 0)),
                pl.BlockSpec((1, c), lambda i: (0, 0))],
      out_specs=pl.BlockSpec((br, c), lambda i: (i, 0)),
      out_shape=jax.ShapeDtypeStruct((np_, c), jnp.float32),
  )(ft, cnt, wout, bout.reshape(1, c))


def kernel(x_, edge_index, edge_weight, n2s_row, n2s_col,
           W0, b0, g0, beta0, W1, b1, g1, beta1, Wout, bout):
  n, c = x_.shape
  e = edge_weight.shape[0]
  nnz = n2s_row.shape[0]

  rpt = ((n + _NW - 1) // _NW + 63) // 64 * 64
  np_ = rpt * _NW
  ch = 2560
  assert e % ch == 0 and (e // ch) >= _NW, e

  xp = jnp.pad(x_, ((0, np_ - n), (0, 0)))
  base, swap = _transforms(xp, W0, b0, g0, beta0, W1, b1, g1, beta1, 1024)

  nnz_pad = (nnz + _L - 1) // _L * _L
  r_pad = jnp.pad(n2s_row.astype(jnp.int32), (0, nnz_pad - nnz),
                  constant_values=-1)
  c_pad = jnp.pad(n2s_col.astype(jnp.int32), (0, nnz_pad - nnz))
  xf = _mix_overwrite(base, swap, r_pad, c_pad, rpt)

  xT3 = jnp.transpose(xf).reshape(_NW, c // _NW, np_)
  nch = e // ch
  meta = jnp.stack([
      edge_index[0].astype(jnp.int32).reshape(nch, ch),
      edge_index[1].astype(jnp.int32).reshape(nch, ch),
      jax.lax.bitcast_convert_type(edge_weight, jnp.int32).reshape(nch, ch),
  ], axis=1).reshape(-1)
  acc = _spmm_t(xT3, meta, np_, ch)

  ft = acc[:, : c // _NW, :].reshape(c, np_)
  cnt = acc[:, c // _NW, :]
  out = _out_linear(ft, cnt, Wout, bout, 1024)
  return out[:n]


# speedup vs baseline: 3.2928x; 1.1368x over previous
import functools

import jax
import jax.numpy as jnp
from jax import lax
from jax.experimental import pallas as pl
from jax.experimental.pallas import tpu as pltpu
from jax.experimental.pallas import tpu_sc as plsc

_NC = 2
_NS = 16
_NW = _NC * _NS
_L = 16
_ZR = 0.8


def _trans_body(x_ref, w0_ref, b0_ref, g0_ref, t0_ref, w1_ref, b1_ref, g1_ref,
                t1_ref, base_ref, swap_ref):
  x = x_ref[...]

  def tr(w, b, g, t):
    h = jnp.dot(x, w, preferred_element_type=jnp.float32) + b
    m = jnp.mean(h, axis=-1, keepdims=True)
    v = jnp.mean((h - m) * (h - m), axis=-1, keepdims=True)
    h = (h - m) / jnp.sqrt(v + 1e-5) * g + t
    return jnp.maximum(h, 0.0)

  x0 = tr(w0_ref[...], b0_ref[...], g0_ref[...], t0_ref[...])
  x1 = tr(w1_ref[...], b1_ref[...], g1_ref[...], t1_ref[...])
  base_ref[...] = _ZR * x0 + (1.0 - _ZR) * x1
  swap_ref[...] = _ZR * x1 + (1.0 - _ZR) * x0


def _transforms(xp, w0, b0, g0, t0, w1, b1, g1, t1, br):
  np_, c = xp.shape
  grid = (np_ // br,)
  row_spec = pl.BlockSpec((br, c), lambda i: (i, 0))
  mat_spec = pl.BlockSpec((c, c), lambda i: (0, 0))
  vec_spec = pl.BlockSpec((1, c), lambda i: (0, 0))
  out_sds = jax.ShapeDtypeStruct((np_, c), jnp.float32)
  return pl.pallas_call(
      _trans_body,
      grid=grid,
      in_specs=[row_spec, mat_spec, vec_spec, vec_spec, vec_spec,
                mat_spec, vec_spec, vec_spec, vec_spec],
      out_specs=[row_spec, row_spec],
      out_shape=[out_sds, out_sds],
  )(xp, w0, b0.reshape(1, c), g0.reshape(1, c), t0.reshape(1, c),
    w1, b1.reshape(1, c), g1.reshape(1, c), t1.reshape(1, c))


def _mix_overwrite(base, swap, r_pad, c_pad, rpt):
  np_, c = base.shape
  ngrp = r_pad.shape[0] // _L
  mesh = plsc.VectorSubcoreMesh(core_axis_name="c", subcore_axis_name="s")

  @functools.partial(
      pl.kernel,
      out_type=jax.ShapeDtypeStruct((np_, c), jnp.float32),
      mesh=mesh,
      compiler_params=pltpu.CompilerParams(needs_layout_passes=False),
      scratch_types=[
          pltpu.VMEM((rpt, c), jnp.float32),
          pltpu.VMEM((rpt, c), jnp.float32),
          pltpu.VMEM((r_pad.shape[0],), jnp.int32),
          pltpu.VMEM((r_pad.shape[0],), jnp.int32),
      ],
  )
  def k(base_hbm, swap_hbm, r_hbm, c_hbm, out_hbm, xloc, sloc, rbuf, cbuf):
    cid = lax.axis_index("c")
    sid = lax.axis_index("s")
    wid = cid * _NS + sid
    row0 = wid * rpt
    pltpu.sync_copy(base_hbm.at[pl.ds(row0, rpt)], xloc)
    pltpu.sync_copy(swap_hbm.at[pl.ds(row0, rpt)], sloc)
    pltpu.sync_copy(r_hbm, rbuf)
    pltpu.sync_copy(c_hbm, cbuf)

    def body(g, carry):
      r16 = rbuf[pl.ds(g * _L, _L)]
      c16 = cbuf[pl.ds(g * _L, _L)]
      lr = r16 - row0
      msk = (lr >= 0) & (lr < rpt)
      lrc = jnp.clip(lr, 0, rpt - 1)
      c16c = jnp.clip(c16, 0, c - 1)
      v = plsc.load_gather(sloc, [lrc, c16c], mask=msk)
      plsc.store_scatter(xloc, [lrc, c16c], v, mask=msk)
      return carry

    lax.fori_loop(0, ngrp, body, 0)
    pltpu.sync_copy(xloc, out_hbm.at[pl.ds(row0, rpt)])

  return k(base, swap, r_pad, c_pad)


def _spmm_t(xT3, meta, np_, ch):
  ncol = xT3.shape[1]
  nch = meta.shape[0] // (3 * ch)
  npg = ch // _L
  mesh = plsc.VectorSubcoreMesh(core_axis_name="c", subcore_axis_name="s")

  @functools.partial(
      pl.kernel,
      out_type=jax.ShapeDtypeStruct((_NW, ncol + 1, np_), jnp.float32),
      mesh=mesh,
      compiler_params=pltpu.CompilerParams(needs_layout_passes=False),
      scratch_types=[
          pltpu.VMEM((ncol, np_), jnp.float32),
          pltpu.VMEM((ncol + 1, np_), jnp.float32),
          pltpu.VMEM((3 * ch,), jnp.int32),
          pltpu.VMEM((3 * ch,), jnp.int32),
          pltpu.SemaphoreType.DMA,
          pltpu.SemaphoreType.DMA,
      ],
  )
  def k(x_hbm, m_hbm, out_hbm, xs, acc, mb0, mb1, sem0, sem1):
    cid = lax.axis_index("c")
    sid = lax.axis_index("s")
    wid = cid * _NS + sid
    pltpu.sync_copy(x_hbm.at[wid], xs)

    zv = jnp.zeros((_L,), jnp.float32)

    def zb(i, carry):
      for j in range(ncol + 1):
        acc[j, pl.ds(i * _L, _L)] = zv
      return carry

    lax.fori_loop(0, np_ // _L, zb, 0)

    ones16 = jnp.full((_L,), 1.0, jnp.float32)
    cnt_row = jnp.full((_L,), ncol, jnp.int32)

    def start(i, mb, sem):
      kc = (i + wid) % nch
      pltpu.async_copy(m_hbm.at[pl.ds(kc * 3 * ch, 3 * ch)], mb, sem)

    def wait(mb, sem):
      pltpu.make_async_copy(m_hbm.at[pl.ds(0, 3 * ch)], mb, sem).wait()

    def make_grp(mb):
      def grp(g, carry):
        r16 = mb[pl.ds(g * _L, _L)]
        c16 = mb[pl.ds(ch + g * _L, _L)]
        w16 = plsc.bitcast(mb[pl.ds(2 * ch + g * _L, _L)], jnp.float32)
        for j in range(ncol):
          jv = jnp.full((_L,), j, jnp.int32)
          v = plsc.load_gather(xs, [jv, c16])
          plsc.addupdate_scatter(acc, [jv, r16], v * w16)
        return carry
      return grp

    start(0, mb0, sem0)

    def chunk2(j, carry):
      i0 = 2 * j
      wait(mb0, sem0)
      start(i0 + 1, mb1, sem1)
      lax.fori_loop(0, npg, make_grp(mb0), 0, unroll=8)
      wait(mb1, sem1)
      start((i0 + 2) % nch, mb0, sem0)
      lax.fori_loop(0, npg, make_grp(mb1), 0, unroll=8)
      return carry

    lax.fori_loop(0, nch // 2, chunk2, 0)
    wait(mb0, sem0)

    def cgrp(g, carry):
      r16 = mb0[pl.ds(g * _L, _L)]
      plsc.addupdate_scatter(acc, [cnt_row, r16], ones16)
      return carry

    def cchunk(i2, carry):
      kc = wid + i2 * _NW

      @pl.when(kc < nch)
      def _():
        pltpu.sync_copy(m_hbm.at[pl.ds(kc * 3 * ch, 3 * ch)], mb0)
        lax.fori_loop(0, npg, cgrp, 0, unroll=8)

      return carry

    lax.fori_loop(0, (nch + _NW - 1) // _NW, cchunk, 0)
    pltpu.sync_copy(acc, out_hbm.at[wid])

  return k(xT3, meta)


def _out_body(ft_ref, cnt_ref, w_ref, b_ref, o_ref):
  n = jnp.sum(cnt_ref[...], axis=0, keepdims=True)
  aggt = ft_ref[...] / jnp.maximum(n, 1.0)
  o_ref[...] = jnp.maximum(
      jax.lax.dot_general(aggt, w_ref[...], (((0,), (0,)), ((), ())),
                          preferred_element_type=jnp.float32) + b_ref[...],
      0.0)


def _out_linear(ft, cnt, wout, bout, br):
  c, np_ = ft.shape
  grid = (np_ // br,)
  return pl.pallas_call(
      _out_body,
      grid=grid,
      in_specs=[pl.BlockSpec((c, br), lambda i: (0, i)),
                pl.BlockSpec((_NW, br), lambda i: (0, i)),
                pl.BlockSpec((c, c), lambda i: (0, 0)),
                pl.BlockSpec((1, c), lambda i: (0, 0))],
      out_specs=pl.BlockSpec((br, c), lambda i: (i, 0)),
      out_shape=jax.ShapeDtypeStruct((np_, c), jnp.float32),
  )(ft, cnt, wout, bout.reshape(1, c))


def kernel(x_, edge_index, edge_weight, n2s_row, n2s_col,
           W0, b0, g0, beta0, W1, b1, g1, beta1, Wout, bout):
  n, c = x_.shape
  e = edge_weight.shape[0]
  nnz = n2s_row.shape[0]

  rpt = ((n + _NW - 1) // _NW + 63) // 64 * 64
  np_ = rpt * _NW
  ch = 1280
  assert e % (2 * ch) == 0 and (e // ch) >= _NW, e

  xp = jnp.pad(x_, ((0, np_ - n), (0, 0)))
  base, swap = _transforms(xp, W0, b0, g0, beta0, W1, b1, g1, beta1, 1024)

  nnz_pad = (nnz + _L - 1) // _L * _L
  r_pad = jnp.pad(n2s_row.astype(jnp.int32), (0, nnz_pad - nnz),
                  constant_values=-1)
  c_pad = jnp.pad(n2s_col.astype(jnp.int32), (0, nnz_pad - nnz))
  xf = _mix_overwrite(base, swap, r_pad, c_pad, rpt)

  xT3 = jnp.transpose(xf).reshape(_NW, c // _NW, np_)
  nch = e // ch
  meta = jnp.stack([
      edge_index[0].astype(jnp.int32).reshape(nch, ch),
      edge_index[1].astype(jnp.int32).reshape(nch, ch),
      jax.lax.bitcast_convert_type(edge_weight, jnp.int32).reshape(nch, ch),
  ], axis=1).reshape(-1)
  acc = _spmm_t(xT3, meta, np_, ch)

  ft = acc[:, : c // _NW, :].reshape(c, np_)
  cnt = acc[:, c // _NW, :]
  out = _out_linear(ft, cnt, Wout, bout, 1024)
  return out[:n]


# speedup vs baseline: 7.3900x; 2.2443x over previous
import functools

import jax
import jax.numpy as jnp
from jax import lax
from jax.experimental import pallas as pl
from jax.experimental.pallas import tpu as pltpu
from jax.experimental.pallas import tpu_sc as plsc

_NC = 2
_NS = 16
_NW = _NC * _NS
_L = 16
_ZR = 0.8


def _trans_body(x_ref, w0_ref, b0_ref, g0_ref, t0_ref, w1_ref, b1_ref, g1_ref,
                t1_ref, base_ref, swap_ref):
  x = x_ref[...]

  def tr(w, b, g, t):
    h = jnp.dot(x, w, preferred_element_type=jnp.float32) + b
    m = jnp.mean(h, axis=-1, keepdims=True)
    v = jnp.mean((h - m) * (h - m), axis=-1, keepdims=True)
    h = (h - m) / jnp.sqrt(v + 1e-5) * g + t
    return jnp.maximum(h, 0.0)

  x0 = tr(w0_ref[...], b0_ref[...], g0_ref[...], t0_ref[...])
  x1 = tr(w1_ref[...], b1_ref[...], g1_ref[...], t1_ref[...])
  base_ref[...] = _ZR * x0 + (1.0 - _ZR) * x1
  swap_ref[...] = _ZR * x1 + (1.0 - _ZR) * x0


def _transforms(xp, w0, b0, g0, t0, w1, b1, g1, t1, br):
  np_, c = xp.shape
  grid = (np_ // br,)
  row_spec = pl.BlockSpec((br, c), lambda i: (i, 0))
  mat_spec = pl.BlockSpec((c, c), lambda i: (0, 0))
  vec_spec = pl.BlockSpec((1, c), lambda i: (0, 0))
  out_sds = jax.ShapeDtypeStruct((np_, c), jnp.float32)
  return pl.pallas_call(
      _trans_body,
      grid=grid,
      in_specs=[row_spec, mat_spec, vec_spec, vec_spec, vec_spec,
                mat_spec, vec_spec, vec_spec, vec_spec],
      out_specs=[row_spec, row_spec],
      out_shape=[out_sds, out_sds],
  )(xp, w0, b0.reshape(1, c), g0.reshape(1, c), t0.reshape(1, c),
    w1, b1.reshape(1, c), g1.reshape(1, c), t1.reshape(1, c))


def _mix_overwrite(base, swap, r_pad, c_pad, rpt):
  np_, c = base.shape
  ngrp = r_pad.shape[0] // _L
  mesh = plsc.VectorSubcoreMesh(core_axis_name="c", subcore_axis_name="s")

  @functools.partial(
      pl.kernel,
      out_type=jax.ShapeDtypeStruct((np_, c), jnp.float32),
      mesh=mesh,
      compiler_params=pltpu.CompilerParams(needs_layout_passes=False),
      scratch_types=[
          pltpu.VMEM((rpt, c), jnp.float32),
          pltpu.VMEM((rpt, c), jnp.float32),
          pltpu.VMEM((r_pad.shape[0],), jnp.int32),
          pltpu.VMEM((r_pad.shape[0],), jnp.int32),
      ],
  )
  def k(base_hbm, swap_hbm, r_hbm, c_hbm, out_hbm, xloc, sloc, rbuf, cbuf):
    cid = lax.axis_index("c")
    sid = lax.axis_index("s")
    wid = cid * _NS + sid
    row0 = wid * rpt
    pltpu.sync_copy(base_hbm.at[pl.ds(row0, rpt)], xloc)
    pltpu.sync_copy(swap_hbm.at[pl.ds(row0, rpt)], sloc)
    pltpu.sync_copy(r_hbm, rbuf)
    pltpu.sync_copy(c_hbm, cbuf)

    def body(g, carry):
      r16 = rbuf[pl.ds(g * _L, _L)]
      c16 = cbuf[pl.ds(g * _L, _L)]
      lr = r16 - row0
      msk = (lr >= 0) & (lr < rpt)
      lrc = jnp.clip(lr, 0, rpt - 1)
      c16c = jnp.clip(c16, 0, c - 1)
      v = plsc.load_gather(sloc, [lrc, c16c], mask=msk)
      plsc.store_scatter(xloc, [lrc, c16c], v, mask=msk)
      return carry

    lax.fori_loop(0, ngrp, body, 0)
    pltpu.sync_copy(xloc, out_hbm.at[pl.ds(row0, rpt)])

  return k(base, swap, r_pad, c_pad)


def _spmm_t(xT3, meta, np_, ch):
  ncol = xT3.shape[1]
  nch = meta.shape[0] // (3 * ch)
  npg = ch // _L
  mesh = plsc.VectorSubcoreMesh(core_axis_name="c", subcore_axis_name="s")

  @functools.partial(
      pl.kernel,
      out_type=jax.ShapeDtypeStruct((_NW, ncol + 1, np_), jnp.float32),
      mesh=mesh,
      compiler_params=pltpu.CompilerParams(needs_layout_passes=False),
      scratch_types=[
          pltpu.VMEM((ncol, np_), jnp.float32),
          pltpu.VMEM((ncol + 1, np_), jnp.float32),
          pltpu.VMEM((3 * ch,), jnp.int32),
          pltpu.VMEM((3 * ch,), jnp.int32),
          pltpu.SemaphoreType.DMA,
          pltpu.SemaphoreType.DMA,
      ],
  )
  def k(x_hbm, m_hbm, out_hbm, xs, acc, mb0, mb1, sem0, sem1):
    cid = lax.axis_index("c")
    sid = lax.axis_index("s")
    wid = cid * _NS + sid
    pltpu.sync_copy(x_hbm.at[wid], xs)

    zv = jnp.zeros((_L,), jnp.float32)

    def zb(i, carry):
      for j in range(ncol + 1):
        acc[j, pl.ds(i * _L, _L)] = zv
      return carry

    lax.fori_loop(0, np_ // _L, zb, 0)

    ones16 = jnp.full((_L,), 1.0, jnp.float32)
    cnt_row = jnp.full((_L,), ncol, jnp.int32)

    def start(i, mb, sem):
      kc = (i + wid) % nch
      pltpu.async_copy(m_hbm.at[pl.ds(kc * 3 * ch, 3 * ch)], mb, sem)

    def wait(mb, sem):
      pltpu.make_async_copy(m_hbm.at[pl.ds(0, 3 * ch)], mb, sem).wait()

    def run_groups(mb):
      @plsc.parallel_loop(0, npg, unroll=8)
      def grp(g):
        r16 = mb[pl.ds(g * _L, _L)]
        c16 = mb[pl.ds(ch + g * _L, _L)]
        w16 = plsc.bitcast(mb[pl.ds(2 * ch + g * _L, _L)], jnp.float32)
        vs = []
        for j in range(ncol):
          jv = jnp.full((_L,), j, jnp.int32)
          vs.append(plsc.load_gather(xs, [jv, c16]) * w16)
        for j in range(ncol):
          jv = jnp.full((_L,), j, jnp.int32)
          plsc.addupdate_scatter(acc, [jv, r16], vs[j])

    start(0, mb0, sem0)

    def chunk2(j, carry):
      i0 = 2 * j
      wait(mb0, sem0)
      start(i0 + 1, mb1, sem1)
      run_groups(mb0)
      wait(mb1, sem1)
      start((i0 + 2) % nch, mb0, sem0)
      run_groups(mb1)
      return carry

    lax.fori_loop(0, nch // 2, chunk2, 0)
    wait(mb0, sem0)

    def cgrp(g, carry):
      r16 = mb0[pl.ds(g * _L, _L)]
      plsc.addupdate_scatter(acc, [cnt_row, r16], ones16)
      return carry

    def cchunk(i2, carry):
      kc = wid + i2 * _NW

      @pl.when(kc < nch)
      def _():
        pltpu.sync_copy(m_hbm.at[pl.ds(kc * 3 * ch, 3 * ch)], mb0)
        lax.fori_loop(0, npg, cgrp, 0, unroll=8)

      return carry

    lax.fori_loop(0, (nch + _NW - 1) // _NW, cchunk, 0)
    pltpu.sync_copy(acc, out_hbm.at[wid])

  return k(xT3, meta)


def _out_body(ft_ref, cnt_ref, w_ref, b_ref, o_ref):
  n = jnp.sum(cnt_ref[...], axis=0, keepdims=True)
  aggt = ft_ref[...] / jnp.maximum(n, 1.0)
  o_ref[...] = jnp.maximum(
      jax.lax.dot_general(aggt, w_ref[...], (((0,), (0,)), ((), ())),
                          preferred_element_type=jnp.float32) + b_ref[...],
      0.0)


def _out_linear(ft, cnt, wout, bout, br):
  c, np_ = ft.shape
  grid = (np_ // br,)
  return pl.pallas_call(
      _out_body,
      grid=grid,
      in_specs=[pl.BlockSpec((c, br), lambda i: (0, i)),
                pl.BlockSpec((_NW, br), lambda i: (0, i)),
                pl.BlockSpec((c, c), lambda i: (0, 0)),
                pl.BlockSpec((1, c), lambda i: (0, 0))],
      out_specs=pl.BlockSpec((br, c), lambda i: (i, 0)),
      out_shape=jax.ShapeDtypeStruct((np_, c), jnp.float32),
  )(ft, cnt, wout, bout.reshape(1, c))


def kernel(x_, edge_index, edge_weight, n2s_row, n2s_col,
           W0, b0, g0, beta0, W1, b1, g1, beta1, Wout, bout):
  n, c = x_.shape
  e = edge_weight.shape[0]
  nnz = n2s_row.shape[0]

  rpt = ((n + _NW - 1) // _NW + 63) // 64 * 64
  np_ = rpt * _NW
  ch = 1280
  assert e % (2 * ch) == 0 and (e // ch) >= _NW, e

  xp = jnp.pad(x_, ((0, np_ - n), (0, 0)))
  base, swap = _transforms(xp, W0, b0, g0, beta0, W1, b1, g1, beta1, 1024)

  nnz_pad = (nnz + _L - 1) // _L * _L
  r_pad = jnp.pad(n2s_row.astype(jnp.int32), (0, nnz_pad - nnz),
                  constant_values=-1)
  c_pad = jnp.pad(n2s_col.astype(jnp.int32), (0, nnz_pad - nnz))
  xf = _mix_overwrite(base, swap, r_pad, c_pad, rpt)

  xT3 = jnp.transpose(xf).reshape(_NW, c // _NW, np_)
  nch = e // ch
  meta = jnp.stack([
      edge_index[0].astype(jnp.int32).reshape(nch, ch),
      edge_index[1].astype(jnp.int32).reshape(nch, ch),
      jax.lax.bitcast_convert_type(edge_weight, jnp.int32).reshape(nch, ch),
  ], axis=1).reshape(-1)
  acc = _spmm_t(xT3, meta, np_, ch)

  ft = acc[:, : c // _NW, :].reshape(c, np_)
  cnt = acc[:, c // _NW, :]
  out = _out_linear(ft, cnt, Wout, bout, 1024)
  return out[:n]


# speedup vs baseline: 7.4924x; 1.0139x over previous
import functools

import jax
import jax.numpy as jnp
from jax import lax
from jax.experimental import pallas as pl
from jax.experimental.pallas import tpu as pltpu
from jax.experimental.pallas import tpu_sc as plsc

_NC = 2
_NS = 16
_NW = _NC * _NS
_L = 16
_ZR = 0.8


def _trans_body(x_ref, w0_ref, b0_ref, g0_ref, t0_ref, w1_ref, b1_ref, g1_ref,
                t1_ref, base_ref, swap_ref):
  x = x_ref[...]

  def tr(w, b, g, t):
    h = jnp.dot(x, w, preferred_element_type=jnp.float32) + b
    m = jnp.mean(h, axis=-1, keepdims=True)
    v = jnp.mean((h - m) * (h - m), axis=-1, keepdims=True)
    h = (h - m) / jnp.sqrt(v + 1e-5) * g + t
    return jnp.maximum(h, 0.0)

  x0 = tr(w0_ref[...], b0_ref[...], g0_ref[...], t0_ref[...])
  x1 = tr(w1_ref[...], b1_ref[...], g1_ref[...], t1_ref[...])
  base_ref[...] = _ZR * x0 + (1.0 - _ZR) * x1
  swap_ref[...] = _ZR * x1 + (1.0 - _ZR) * x0


def _transforms(xp, w0, b0, g0, t0, w1, b1, g1, t1, br):
  np_, c = xp.shape
  grid = (np_ // br,)
  row_spec = pl.BlockSpec((br, c), lambda i: (i, 0))
  mat_spec = pl.BlockSpec((c, c), lambda i: (0, 0))
  vec_spec = pl.BlockSpec((1, c), lambda i: (0, 0))
  out_sds = jax.ShapeDtypeStruct((np_, c), jnp.float32)
  return pl.pallas_call(
      _trans_body,
      grid=grid,
      in_specs=[row_spec, mat_spec, vec_spec, vec_spec, vec_spec,
                mat_spec, vec_spec, vec_spec, vec_spec],
      out_specs=[row_spec, row_spec],
      out_shape=[out_sds, out_sds],
  )(xp, w0, b0.reshape(1, c), g0.reshape(1, c), t0.reshape(1, c),
    w1, b1.reshape(1, c), g1.reshape(1, c), t1.reshape(1, c))


def _mix_overwrite(base, swap, r_pad, c_pad, rpt):
  np_, c = base.shape
  ngrp = r_pad.shape[0] // _L
  mesh = plsc.VectorSubcoreMesh(core_axis_name="c", subcore_axis_name="s")

  @functools.partial(
      pl.kernel,
      out_type=jax.ShapeDtypeStruct((np_, c), jnp.float32),
      mesh=mesh,
      compiler_params=pltpu.CompilerParams(needs_layout_passes=False),
      scratch_types=[
          pltpu.VMEM((rpt, c), jnp.float32),
          pltpu.VMEM((rpt, c), jnp.float32),
          pltpu.VMEM((r_pad.shape[0],), jnp.int32),
          pltpu.VMEM((r_pad.shape[0],), jnp.int32),
      ],
  )
  def k(base_hbm, swap_hbm, r_hbm, c_hbm, out_hbm, xloc, sloc, rbuf, cbuf):
    cid = lax.axis_index("c")
    sid = lax.axis_index("s")
    wid = cid * _NS + sid
    row0 = wid * rpt
    pltpu.sync_copy(base_hbm.at[pl.ds(row0, rpt)], xloc)
    pltpu.sync_copy(swap_hbm.at[pl.ds(row0, rpt)], sloc)
    pltpu.sync_copy(r_hbm, rbuf)
    pltpu.sync_copy(c_hbm, cbuf)

    def body(g, carry):
      r16 = rbuf[pl.ds(g * _L, _L)]
      c16 = cbuf[pl.ds(g * _L, _L)]
      lr = r16 - row0
      msk = (lr >= 0) & (lr < rpt)
      lrc = jnp.clip(lr, 0, rpt - 1)
      c16c = jnp.clip(c16, 0, c - 1)
      v = plsc.load_gather(sloc, [lrc, c16c], mask=msk)
      plsc.store_scatter(xloc, [lrc, c16c], v, mask=msk)
      return carry

    lax.fori_loop(0, ngrp, body, 0)
    pltpu.sync_copy(xloc, out_hbm.at[pl.ds(row0, rpt)])

  return k(base, swap, r_pad, c_pad)


def _spmm_t(xT3, meta, np_, ch):
  ncol = xT3.shape[1]
  nch = meta.shape[0] // (3 * ch)
  npg = ch // _L
  mesh = plsc.VectorSubcoreMesh(core_axis_name="c", subcore_axis_name="s")

  @functools.partial(
      pl.kernel,
      out_type=jax.ShapeDtypeStruct((_NW, ncol + 1, np_), jnp.float32),
      mesh=mesh,
      compiler_params=pltpu.CompilerParams(needs_layout_passes=False),
      scratch_types=[
          pltpu.VMEM((ncol, np_), jnp.float32),
          pltpu.VMEM((ncol + 1, np_), jnp.float32),
          pltpu.VMEM((3 * ch,), jnp.int32),
          pltpu.VMEM((3 * ch,), jnp.int32),
          pltpu.SemaphoreType.DMA,
          pltpu.SemaphoreType.DMA,
      ],
  )
  def k(x_hbm, m_hbm, out_hbm, xs, acc, mb0, mb1, sem0, sem1):
    cid = lax.axis_index("c")
    sid = lax.axis_index("s")
    wid = cid * _NS + sid
    pltpu.sync_copy(x_hbm.at[wid], xs)

    zv = jnp.zeros((_L,), jnp.float32)

    @plsc.parallel_loop(0, np_ // _L, unroll=4)
    def zb(i):
      for j in range(ncol + 1):
        acc[j, pl.ds(i * _L, _L)] = zv

    ones16 = jnp.full((_L,), 1.0, jnp.float32)
    cnt_row = jnp.full((_L,), ncol, jnp.int32)

    def start(i, mb, sem):
      kc = (i + wid) % nch
      pltpu.async_copy(m_hbm.at[pl.ds(kc * 3 * ch, 3 * ch)], mb, sem)

    def wait(mb, sem):
      pltpu.make_async_copy(m_hbm.at[pl.ds(0, 3 * ch)], mb, sem).wait()

    def run_groups(mb):
      @plsc.parallel_loop(0, npg, unroll=8)
      def grp(g):
        r16 = mb[pl.ds(g * _L, _L)]
        c16 = mb[pl.ds(ch + g * _L, _L)]
        w16 = plsc.bitcast(mb[pl.ds(2 * ch + g * _L, _L)], jnp.float32)
        vs = []
        for j in range(ncol):
          jv = jnp.full((_L,), j, jnp.int32)
          vs.append(plsc.load_gather(xs, [jv, c16]) * w16)
        for j in range(ncol):
          jv = jnp.full((_L,), j, jnp.int32)
          plsc.addupdate_scatter(acc, [jv, r16], vs[j])

    start(0, mb0, sem0)

    def chunk2(j, carry):
      i0 = 2 * j
      wait(mb0, sem0)
      start(i0 + 1, mb1, sem1)
      run_groups(mb0)
      wait(mb1, sem1)
      start((i0 + 2) % nch, mb0, sem0)
      run_groups(mb1)
      return carry

    lax.fori_loop(0, nch // 2, chunk2, 0)
    wait(mb0, sem0)

    def cchunk(i2, carry):
      kc = wid + i2 * _NW

      @pl.when(kc < nch)
      def _():
        pltpu.sync_copy(m_hbm.at[pl.ds(kc * 3 * ch, 3 * ch)], mb0)

        @plsc.parallel_loop(0, npg, unroll=8)
        def cgrp(g):
          r16 = mb0[pl.ds(g * _L, _L)]
          plsc.addupdate_scatter(acc, [cnt_row, r16], ones16)

      return carry

    lax.fori_loop(0, (nch + _NW - 1) // _NW, cchunk, 0)
    pltpu.sync_copy(acc, out_hbm.at[wid])

  return k(xT3, meta)


def _out_body(ft_ref, cnt_ref, w_ref, b_ref, o_ref):
  n = jnp.sum(cnt_ref[...], axis=0, keepdims=True)
  aggt = ft_ref[...] / jnp.maximum(n, 1.0)
  o_ref[...] = jnp.maximum(
      jax.lax.dot_general(aggt, w_ref[...], (((0,), (0,)), ((), ())),
                          preferred_element_type=jnp.float32) + b_ref[...],
      0.0)


def _out_linear(ft, cnt, wout, bout, br):
  c, np_ = ft.shape
  grid = (np_ // br,)
  return pl.pallas_call(
      _out_body,
      grid=grid,
      in_specs=[pl.BlockSpec((c, br), lambda i: (0, i)),
                pl.BlockSpec((_NW, br), lambda i: (0, i)),
                pl.BlockSpec((c, c), lambda i: (0, 0)),
                pl.BlockSpec((1, c), lambda i: (0, 0))],
      out_specs=pl.BlockSpec((br, c), lambda i: (i, 0)),
      out_shape=jax.ShapeDtypeStruct((np_, c), jnp.float32),
  )(ft, cnt, wout, bout.reshape(1, c))


def kernel(x_, edge_index, edge_weight, n2s_row, n2s_col,
           W0, b0, g0, beta0, W1, b1, g1, beta1, Wout, bout):
  n, c = x_.shape
  e = edge_weight.shape[0]
  nnz = n2s_row.shape[0]

  rpt = ((n + _NW - 1) // _NW + 63) // 64 * 64
  np_ = rpt * _NW
  ch = 1280
  assert e % (2 * ch) == 0 and (e // ch) >= _NW, e

  xp = jnp.pad(x_, ((0, np_ - n), (0, 0)))
  base, swap = _transforms(xp, W0, b0, g0, beta0, W1, b1, g1, beta1, 1024)

  nnz_pad = (nnz + _L - 1) // _L * _L
  r_pad = jnp.pad(n2s_row.astype(jnp.int32), (0, nnz_pad - nnz),
                  constant_values=-1)
  c_pad = jnp.pad(n2s_col.astype(jnp.int32), (0, nnz_pad - nnz))
  xf = _mix_overwrite(base, swap, r_pad, c_pad, rpt)

  xT3 = jnp.transpose(xf).reshape(_NW, c // _NW, np_)
  nch = e // ch
  meta = jnp.stack([
      edge_index[0].astype(jnp.int32).reshape(nch, ch),
      edge_index[1].astype(jnp.int32).reshape(nch, ch),
      jax.lax.bitcast_convert_type(edge_weight, jnp.int32).reshape(nch, ch),
  ], axis=1).reshape(-1)
  acc = _spmm_t(xT3, meta, np_, ch)

  ft = acc[:, : c // _NW, :].reshape(c, np_)
  cnt = acc[:, c // _NW, :]
  out = _out_linear(ft, cnt, Wout, bout, 1024)
  return out[:n]


# speedup vs baseline: 7.6922x; 1.0267x over previous
import functools

import jax
import jax.numpy as jnp
from jax import lax
from jax.experimental import pallas as pl
from jax.experimental.pallas import tpu as pltpu
from jax.experimental.pallas import tpu_sc as plsc

_NC = 2
_NS = 16
_NW = _NC * _NS
_L = 16
_ZR = 0.8


def _trans_body(x_ref, w0_ref, b0_ref, g0_ref, t0_ref, w1_ref, b1_ref, g1_ref,
                t1_ref, base_ref, swap_ref):
  x = x_ref[...]

  def tr(w, b, g, t):
    h = jnp.dot(x, w, preferred_element_type=jnp.float32) + b
    m = jnp.mean(h, axis=-1, keepdims=True)
    v = jnp.mean((h - m) * (h - m), axis=-1, keepdims=True)
    h = (h - m) / jnp.sqrt(v + 1e-5) * g + t
    return jnp.maximum(h, 0.0)

  x0 = tr(w0_ref[...], b0_ref[...], g0_ref[...], t0_ref[...])
  x1 = tr(w1_ref[...], b1_ref[...], g1_ref[...], t1_ref[...])
  base_ref[...] = _ZR * x0 + (1.0 - _ZR) * x1
  swap_ref[...] = _ZR * x1 + (1.0 - _ZR) * x0


def _transforms(xp, w0, b0, g0, t0, w1, b1, g1, t1, br):
  np_, c = xp.shape
  grid = (np_ // br,)
  row_spec = pl.BlockSpec((br, c), lambda i: (i, 0))
  mat_spec = pl.BlockSpec((c, c), lambda i: (0, 0))
  vec_spec = pl.BlockSpec((1, c), lambda i: (0, 0))
  out_sds = jax.ShapeDtypeStruct((np_, c), jnp.float32)
  return pl.pallas_call(
      _trans_body,
      grid=grid,
      in_specs=[row_spec, mat_spec, vec_spec, vec_spec, vec_spec,
                mat_spec, vec_spec, vec_spec, vec_spec],
      out_specs=[row_spec, row_spec],
      out_shape=[out_sds, out_sds],
  )(xp, w0, b0.reshape(1, c), g0.reshape(1, c), t0.reshape(1, c),
    w1, b1.reshape(1, c), g1.reshape(1, c), t1.reshape(1, c))


def _mix_overwrite(base, swap, r_pad, c_pad, rpt):
  np_, c = base.shape
  ngrp = r_pad.shape[0] // _L
  mesh = plsc.VectorSubcoreMesh(core_axis_name="c", subcore_axis_name="s")

  @functools.partial(
      pl.kernel,
      out_type=jax.ShapeDtypeStruct((np_, c), jnp.float32),
      mesh=mesh,
      compiler_params=pltpu.CompilerParams(needs_layout_passes=False),
      scratch_types=[
          pltpu.VMEM((rpt, c), jnp.float32),
          pltpu.VMEM((rpt, c), jnp.float32),
          pltpu.VMEM((r_pad.shape[0],), jnp.int32),
          pltpu.VMEM((r_pad.shape[0],), jnp.int32),
      ],
  )
  def k(base_hbm, swap_hbm, r_hbm, c_hbm, out_hbm, xloc, sloc, rbuf, cbuf):
    cid = lax.axis_index("c")
    sid = lax.axis_index("s")
    wid = cid * _NS + sid
    row0 = wid * rpt
    pltpu.sync_copy(base_hbm.at[pl.ds(row0, rpt)], xloc)
    pltpu.sync_copy(swap_hbm.at[pl.ds(row0, rpt)], sloc)
    pltpu.sync_copy(r_hbm, rbuf)
    pltpu.sync_copy(c_hbm, cbuf)

    def body(g, carry):
      r16 = rbuf[pl.ds(g * _L, _L)]
      c16 = cbuf[pl.ds(g * _L, _L)]
      lr = r16 - row0
      msk = (lr >= 0) & (lr < rpt)
      lrc = jnp.clip(lr, 0, rpt - 1)
      c16c = jnp.clip(c16, 0, c - 1)
      v = plsc.load_gather(sloc, [lrc, c16c], mask=msk)
      plsc.store_scatter(xloc, [lrc, c16c], v, mask=msk)
      return carry

    lax.fori_loop(0, ngrp, body, 0)
    pltpu.sync_copy(xloc, out_hbm.at[pl.ds(row0, rpt)])

  return k(base, swap, r_pad, c_pad)


def _spmm_t(xT3, meta, np_, ch):
  ncol = xT3.shape[1]
  nch = meta.shape[0] // (3 * ch)
  npg = ch // _L
  mesh = plsc.VectorSubcoreMesh(core_axis_name="c", subcore_axis_name="s")

  @functools.partial(
      pl.kernel,
      out_type=jax.ShapeDtypeStruct((_NW, ncol + 1, np_), jnp.float32),
      mesh=mesh,
      compiler_params=pltpu.CompilerParams(needs_layout_passes=False),
      scratch_types=[
          pltpu.VMEM((ncol, np_), jnp.float32),
          pltpu.VMEM((ncol + 1, np_), jnp.float32),
          pltpu.VMEM((3 * ch,), jnp.int32),
          pltpu.VMEM((3 * ch,), jnp.int32),
          pltpu.SemaphoreType.DMA,
          pltpu.SemaphoreType.DMA,
      ],
  )
  def k(x_hbm, m_hbm, out_hbm, xs, acc, mb0, mb1, sem0, sem1):
    cid = lax.axis_index("c")
    sid = lax.axis_index("s")
    wid = cid * _NS + sid
    pltpu.sync_copy(x_hbm.at[wid], xs)

    zv = jnp.zeros((_L,), jnp.float32)

    @plsc.parallel_loop(0, np_ // _L, unroll=4)
    def zb(i):
      for j in range(ncol + 1):
        acc[j, pl.ds(i * _L, _L)] = zv

    ones16 = jnp.full((_L,), 1.0, jnp.float32)
    cnt_row = jnp.full((_L,), ncol, jnp.int32)

    def start(i, mb, sem):
      kc = (i + wid) % nch
      pltpu.async_copy(m_hbm.at[pl.ds(kc * 3 * ch, 3 * ch)], mb, sem)

    def wait(mb, sem):
      pltpu.make_async_copy(m_hbm.at[pl.ds(0, 3 * ch)], mb, sem).wait()

    def run_groups(mb):
      @plsc.parallel_loop(0, npg, unroll=16)
      def grp(g):
        r16 = mb[pl.ds(g * _L, _L)]
        c16 = mb[pl.ds(ch + g * _L, _L)]
        w16 = plsc.bitcast(mb[pl.ds(2 * ch + g * _L, _L)], jnp.float32)
        vs = []
        for j in range(ncol):
          jv = jnp.full((_L,), j, jnp.int32)
          vs.append(plsc.load_gather(xs, [jv, c16]) * w16)
        for j in range(ncol):
          jv = jnp.full((_L,), j, jnp.int32)
          plsc.addupdate_scatter(acc, [jv, r16], vs[j])

    start(0, mb0, sem0)

    def chunk2(j, carry):
      i0 = 2 * j
      wait(mb0, sem0)
      start(i0 + 1, mb1, sem1)
      run_groups(mb0)
      wait(mb1, sem1)
      start((i0 + 2) % nch, mb0, sem0)
      run_groups(mb1)
      return carry

    lax.fori_loop(0, nch // 2, chunk2, 0)
    wait(mb0, sem0)

    def cchunk(i2, carry):
      kc = wid + i2 * _NW

      @pl.when(kc < nch)
      def _():
        pltpu.sync_copy(m_hbm.at[pl.ds(kc * 3 * ch, 3 * ch)], mb0)

        @plsc.parallel_loop(0, npg, unroll=8)
        def cgrp(g):
          r16 = mb0[pl.ds(g * _L, _L)]
          plsc.addupdate_scatter(acc, [cnt_row, r16], ones16)

      return carry

    lax.fori_loop(0, (nch + _NW - 1) // _NW, cchunk, 0)
    pltpu.sync_copy(acc, out_hbm.at[wid])

  return k(xT3, meta)


def _out_body(ft_ref, cnt_ref, w_ref, b_ref, o_ref):
  n = jnp.sum(cnt_ref[...], axis=0, keepdims=True)
  aggt = ft_ref[...] / jnp.maximum(n, 1.0)
  o_ref[...] = jnp.maximum(
      jax.lax.dot_general(aggt, w_ref[...], (((0,), (0,)), ((), ())),
                          preferred_element_type=jnp.float32) + b_ref[...],
      0.0)


def _out_linear(ft, cnt, wout, bout, br):
  c, np_ = ft.shape
  grid = (np_ // br,)
  return pl.pallas_call(
      _out_body,
      grid=grid,
      in_specs=[pl.BlockSpec((c, br), lambda i: (0, i)),
                pl.BlockSpec((_NW, br), lambda i: (0, i)),
                pl.BlockSpec((c, c), lambda i: (0, 0)),
                pl.BlockSpec((1, c), lambda i: (0, 0))],
      out_specs=pl.BlockSpec((br, c), lambda i: (i, 0)),
      out_shape=jax.ShapeDtypeStruct((np_, c), jnp.float32),
  )(ft, cnt, wout, bout.reshape(1, c))


def kernel(x_, edge_index, edge_weight, n2s_row, n2s_col,
           W0, b0, g0, beta0, W1, b1, g1, beta1, Wout, bout):
  n, c = x_.shape
  e = edge_weight.shape[0]
  nnz = n2s_row.shape[0]

  rpt = ((n + _NW - 1) // _NW + 63) // 64 * 64
  np_ = rpt * _NW
  ch = 1280
  assert e % (2 * ch) == 0 and (e // ch) >= _NW, e

  xp = jnp.pad(x_, ((0, np_ - n), (0, 0)))
  base, swap = _transforms(xp, W0, b0, g0, beta0, W1, b1, g1, beta1, 1024)

  nnz_pad = (nnz + _L - 1) // _L * _L
  r_pad = jnp.pad(n2s_row.astype(jnp.int32), (0, nnz_pad - nnz),
                  constant_values=-1)
  c_pad = jnp.pad(n2s_col.astype(jnp.int32), (0, nnz_pad - nnz))
  xf = _mix_overwrite(base, swap, r_pad, c_pad, rpt)

  xT3 = jnp.transpose(xf).reshape(_NW, c // _NW, np_)
  nch = e // ch
  meta = jnp.stack([
      edge_index[0].astype(jnp.int32).reshape(nch, ch),
      edge_index[1].astype(jnp.int32).reshape(nch, ch),
      jax.lax.bitcast_convert_type(edge_weight, jnp.int32).reshape(nch, ch),
  ], axis=1).reshape(-1)
  acc = _spmm_t(xT3, meta, np_, ch)

  ft = acc[:, : c // _NW, :].reshape(c, np_)
  cnt = acc[:, c // _NW, :]
  out = _out_linear(ft, cnt, Wout, bout, 1024)
  return out[:n]
